# baseline (speedup 1.0000x reference)
import functools

import jax
import jax.numpy as jnp
from jax import lax
from jax.experimental import pallas as pl
from jax.experimental.pallas import tpu as pltpu
from jax.experimental.pallas import tpu_sc as plsc

N = 10000
E = 320000
D = 128
H = 128
OUT = 128

NW = 32
EPW = E // NW
CHUNK = 80
NCH = EPW // CHUNK
SPT = 640
ZR = 80
BLK = 1000

_GATHER_DNUMS = lax.GatherDimensionNumbers(
    offset_dims=(), collapsed_slice_dims=(0,), start_index_map=(0,))


def _bcast_lane(v, e):
    idx = jnp.full((16, 1), e, jnp.int32)
    return lax.gather(v, idx, _GATHER_DNUMS, slice_sizes=(1,),
                      mode=lax.GatherScatterMode.PROMISE_IN_BOUNDS)


def _sc_agg(h, src, dst2d, ew):
    mesh = plsc.VectorSubcoreMesh(core_axis_name="c", subcore_axis_name="s")

    @functools.partial(
        pl.kernel,
        out_type=jax.ShapeDtypeStruct((2, N, H), jnp.float32),
        mesh=mesh,
        scratch_types=[
            pltpu.VMEM_SHARED((N, H), jnp.float32),
            pltpu.VMEM((EPW,), jnp.int32),
            pltpu.VMEM((NCH, CHUNK), jnp.int32),
            pltpu.VMEM((CHUNK, H), jnp.float32),
            pltpu.VMEM((CHUNK, H), jnp.float32),
            pltpu.VMEM((CHUNK,), jnp.float32),
            pltpu.VMEM((CHUNK,), jnp.float32),
            pltpu.SemaphoreType.DMA,
            pltpu.SemaphoreType.DMA,
            pltpu.SemaphoreType.DMA,
            pltpu.SemaphoreType.DMA,
            pltpu.SemaphoreType.DMA,
            pltpu.SemaphoreType.DMA,
        ],
    )
    def k(h_hbm, src_hbm, dst_hbm, ew_hbm, out_hbm,
          acc, src_v, dst_v, rows_a, rows_b, ew_a, ew_b,
          sem_a, sem_b, sew_a, sew_b, ssc_a, ssc_b):
        cid = lax.axis_index("c")
        sid = lax.axis_index("s")
        wid = cid * 16 + sid
        ebase = wid * EPW
        pltpu.sync_copy(src_hbm.at[pl.ds(ebase, EPW)], src_v)
        pltpu.sync_copy(dst_hbm.at[wid], dst_v)

        z16 = jnp.zeros((16,), jnp.float32)

        def zero_body(r, carry):
            for j in range(8):
                rows_a[r, pl.ds(j * 16, 16)] = z16
            return carry

        lax.fori_loop(0, ZR, zero_body, 0)
        row0 = sid * SPT
        not_last = sid != 15
        for kk in range(SPT // ZR):
            def zcopy(kk=kk):
                pltpu.sync_copy(rows_a, acc.at[pl.ds(row0 + kk * ZR, ZR)])
            if kk < 5:
                zcopy()
            else:
                pl.when(not_last)(zcopy)
        plsc.subcore_barrier()

        def start_gather(c, rows, sem, ew, sew):
            pltpu.async_copy(ew_hbm.at[pl.ds(ebase + c * CHUNK, CHUNK)],
                             ew, sew)
            pltpu.async_copy(
                h_hbm.at[src_v.at[pl.ds(c * CHUNK, CHUNK)]], rows, sem)

        def wait_gather(rows, sem, ew, sew):
            pltpu.make_async_copy(ew_hbm.at[pl.ds(0, CHUNK)], ew, sew).wait()
            pltpu.make_async_copy(
                h_hbm.at[pl.ds(0, CHUNK)], rows, sem).wait()

        def start_scatter(c, rows, ssc):
            pltpu.async_copy(rows, acc.at[dst_v.at[c]], ssc, add=True)

        def wait_scatter(rows, ssc):
            pltpu.make_async_copy(rows, acc.at[pl.ds(0, CHUNK)], ssc).wait()

        def scale(c, rows, ew):
            def grp_body(g, carry2):
                w_g = ew[pl.ds(g * 16, 16)]
                for e in range(16):
                    w16 = _bcast_lane(w_g, e)
                    r = g * 16 + e
                    for j in range(8):
                        rows[r, pl.ds(j * 16, 16)] = (
                            rows[r, pl.ds(j * 16, 16)] * w16)
                return carry2

            lax.fori_loop(0, CHUNK // 16, grp_body, 0)

        start_gather(0, rows_a, sem_a, ew_a, sew_a)
        wait_gather(rows_a, sem_a, ew_a, sew_a)
        scale(0, rows_a, ew_a)
        start_scatter(0, rows_a, ssc_a)
        start_gather(1, rows_b, sem_b, ew_b, sew_b)

        def step(c, rx, ewx, semx, sewx, sscx, ry, ewy, semy, sewy, sscy):
            wait_gather(rx, semx, ewx, sewx)
            scale(c, rx, ewx)
            start_scatter(c, rx, sscx)
            wait_scatter(ry, sscy)
            start_gather(c + 1, ry, semy, ewy, sewy)

        def pair_body(p, carry):
            c0 = 1 + 2 * p
            step(c0, rows_b, ew_b, sem_b, sew_b, ssc_b,
                 rows_a, ew_a, sem_a, sew_a, ssc_a)
            step(c0 + 1, rows_a, ew_a, sem_a, sew_a, ssc_a,
                 rows_b, ew_b, sem_b, sew_b, ssc_b)
            return carry

        lax.fori_loop(0, (NCH - 3) // 2, pair_body, 0)
        step(NCH - 2, rows_b, ew_b, sem_b, sew_b, ssc_b,
             rows_a, ew_a, sem_a, sew_a, ssc_a)
        wait_gather(rows_a, sem_a, ew_a, sew_a)
        scale(NCH - 1, rows_a, ew_a)
        start_scatter(NCH - 1, rows_a, ssc_a)
        wait_scatter(rows_b, ssc_b)
        wait_scatter(rows_a, ssc_a)
        plsc.subcore_barrier()
        for kk in range(SPT // ZR):
            def wcopy(kk=kk):
                r0 = row0 + kk * ZR
                pltpu.sync_copy(acc.at[pl.ds(r0, ZR)],
                                out_hbm.at[cid, pl.ds(r0, ZR)])
            if kk < 5:
                wcopy()
            else:
                pl.when(not_last)(wcopy)

    return k(h, src, dst2d, ew)


def _emb_body(x_ref, w_ref, b_ref, o_ref):
    o_ref[...] = (jnp.dot(x_ref[...], w_ref[...],
                          preferred_element_type=jnp.float32) + b_ref[...])


def _tc_emb(x, W, b):
    return pl.pallas_call(
        _emb_body,
        grid=(N // BLK,),
        in_specs=[pl.BlockSpec((BLK, D), lambda i: (i, 0)),
                  pl.BlockSpec((D, H), lambda i: (0, 0)),
                  pl.BlockSpec((1, H), lambda i: (0, 0))],
        out_specs=pl.BlockSpec((BLK, H), lambda i: (i, 0)),
        out_shape=jax.ShapeDtypeStruct((N, H), jnp.float32),
    )(x, W, b.reshape(1, H))


def _mlp_body(h_ref, a0_ref, a1_ref, w1_ref, b1_ref, w2_ref, b2_ref, o_ref):
    z = h_ref[...] + a0_ref[0] + a1_ref[0]
    t = jnp.maximum(jnp.dot(z, w1_ref[...],
                            preferred_element_type=jnp.float32) + b1_ref[...], 0.0)
    o_ref[...] = jnp.maximum(
        jnp.dot(t, w2_ref[...], preferred_element_type=jnp.float32)
        + b2_ref[...], 0.0)


def _tc_mlp(h, agg, W1, b1, W2, b2):
    nb = N // BLK
    return pl.pallas_call(
        _mlp_body,
        grid=(nb,),
        in_specs=[pl.BlockSpec((BLK, H), lambda i: (i, 0)),
                  pl.BlockSpec((1, BLK, H), lambda i: (0, i, 0)),
                  pl.BlockSpec((1, BLK, H), lambda i: (1, i, 0)),
                  pl.BlockSpec((H, H), lambda i: (0, 0)),
                  pl.BlockSpec((1, H), lambda i: (0, 0)),
                  pl.BlockSpec((H, H), lambda i: (0, 0)),
                  pl.BlockSpec((1, H), lambda i: (0, 0))],
        out_specs=pl.BlockSpec((BLK, H), lambda i: (i, 0)),
        out_shape=jax.ShapeDtypeStruct((N, H), jnp.float32),
    )(h, agg, agg, W1, b1.reshape(1, H), W2, b2.reshape(1, H))


def _mlp_ro_body(h_ref, a0_ref, a1_ref, w1_ref, b1_ref, w2_ref, b2_ref,
                 wr_ref, br_ref, o_ref):
    z = h_ref[...] + a0_ref[0] + a1_ref[0]
    t = jnp.maximum(jnp.dot(z, w1_ref[...],
                            preferred_element_type=jnp.float32) + b1_ref[...], 0.0)
    u = jnp.maximum(
        jnp.dot(t, w2_ref[...], preferred_element_type=jnp.float32)
        + b2_ref[...], 0.0)
    o_ref[...] = (jnp.dot(u, wr_ref[...],
                          preferred_element_type=jnp.float32) + br_ref[...])


def _tc_mlp_ro(h, agg, W1, b1, W2, b2, Wr, br):
    nb = N // BLK
    return pl.pallas_call(
        _mlp_ro_body,
        grid=(nb,),
        in_specs=[pl.BlockSpec((BLK, H), lambda i: (i, 0)),
                  pl.BlockSpec((1, BLK, H), lambda i: (0, i, 0)),
                  pl.BlockSpec((1, BLK, H), lambda i: (1, i, 0)),
                  pl.BlockSpec((H, H), lambda i: (0, 0)),
                  pl.BlockSpec((1, H), lambda i: (0, 0)),
                  pl.BlockSpec((H, H), lambda i: (0, 0)),
                  pl.BlockSpec((1, H), lambda i: (0, 0)),
                  pl.BlockSpec((H, OUT), lambda i: (0, 0)),
                  pl.BlockSpec((1, OUT), lambda i: (0, 0))],
        out_specs=pl.BlockSpec((BLK, OUT), lambda i: (i, 0)),
        out_shape=jax.ShapeDtypeStruct((N, OUT), jnp.float32),
    )(h, agg, agg, W1, b1.reshape(1, H), W2, b2.reshape(1, H),
      Wr, br.reshape(1, OUT))


def kernel(features, edge_index, edge_weight,
           W_emb, b_emb,
           W1_0, b1_0, W2_0, b2_0,
           W1_1, b1_1, W2_1, b2_1,
           W_ro, b_ro):
    src = edge_index[0].astype(jnp.int32)
    dst2d = edge_index[1].astype(jnp.int32).reshape(NW, NCH, CHUNK)
    h0 = _tc_emb(features, W_emb, b_emb)
    agg0 = _sc_agg(h0, src, dst2d, edge_weight)
    h1 = _tc_mlp(h0, agg0, W1_0, b1_0, W2_0, b2_0)
    agg1 = _sc_agg(h1, src, dst2d, edge_weight)
    return _tc_mlp_ro(h1, agg1, W1_1, b1_1, W2_1, b2_1, W_ro, b_ro)


# speedup vs baseline: 1.3627x; 1.3627x over previous
import functools

import jax
import jax.numpy as jnp
from jax import lax
from jax.experimental import pallas as pl
from jax.experimental.pallas import tpu as pltpu
from jax.experimental.pallas import tpu_sc as plsc

N = 10000
E = 320000
D = 128
H = 128
OUT = 128

NW = 32
EPW = E // NW
CHUNK = 80
NCH = EPW // CHUNK
SPT = 640
ZR = 80
BLK = 1000

_GATHER_DNUMS = lax.GatherDimensionNumbers(
    offset_dims=(), collapsed_slice_dims=(0,), start_index_map=(0,))


def _bcast_lane(v, e):
    idx = jnp.full((16, 1), e, jnp.int32)
    return lax.gather(v, idx, _GATHER_DNUMS, slice_sizes=(1,),
                      mode=lax.GatherScatterMode.PROMISE_IN_BOUNDS)


def _sc_agg(h, src, dst2d, ew):
    mesh = plsc.VectorSubcoreMesh(core_axis_name="c", subcore_axis_name="s")

    @functools.partial(
        pl.kernel,
        out_type=jax.ShapeDtypeStruct((2, N, H), jnp.float32),
        mesh=mesh,
        scratch_types=[
            pltpu.VMEM_SHARED((N, H), jnp.float32),
            pltpu.VMEM((EPW,), jnp.int32),
            pltpu.VMEM((NCH, CHUNK), jnp.int32),
            pltpu.VMEM((CHUNK, H), jnp.float32),
            pltpu.VMEM((CHUNK, H), jnp.float32),
            pltpu.VMEM((CHUNK,), jnp.float32),
            pltpu.VMEM((CHUNK,), jnp.float32),
            pltpu.SemaphoreType.DMA,
            pltpu.SemaphoreType.DMA,
            pltpu.SemaphoreType.DMA,
            pltpu.SemaphoreType.DMA,
        ],
    )
    def k(h_hbm, src_hbm, dst_hbm, ew_hbm, out_hbm,
          acc, src_v, dst_v, rows_a, rows_b, ew_a, ew_b,
          sem_a, sem_b, sew_a, sew_b):
        cid = lax.axis_index("c")
        sid = lax.axis_index("s")
        wid = cid * 16 + sid
        ebase = wid * EPW
        pltpu.sync_copy(src_hbm.at[pl.ds(ebase, EPW)], src_v)
        pltpu.sync_copy(dst_hbm.at[wid], dst_v)

        z16 = jnp.zeros((16,), jnp.float32)

        def zero_body(r, carry):
            for j in range(8):
                rows_a[r, pl.ds(j * 16, 16)] = z16
            return carry

        lax.fori_loop(0, ZR, zero_body, 0)
        row0 = sid * SPT
        not_last = sid != 15
        for kk in range(SPT // ZR):
            def zcopy(kk=kk):
                pltpu.sync_copy(rows_a, acc.at[pl.ds(row0 + kk * ZR, ZR)])
            if kk < 5:
                zcopy()
            else:
                pl.when(not_last)(zcopy)
        plsc.subcore_barrier()

        def start_gather(c, rows, sem, ew, sew):
            pltpu.async_copy(ew_hbm.at[pl.ds(ebase + c * CHUNK, CHUNK)],
                             ew, sew)
            pltpu.async_copy(
                h_hbm.at[src_v.at[pl.ds(c * CHUNK, CHUNK)]], rows, sem)

        def wait_gather(rows, sem, ew, sew):
            pltpu.make_async_copy(ew_hbm.at[pl.ds(0, CHUNK)], ew, sew).wait()
            pltpu.make_async_copy(
                h_hbm.at[src_v.at[pl.ds(0, CHUNK)]], rows, sem).wait()

        def scale(c, rows, ew):
            def grp_body(g, carry2):
                w_g = ew[pl.ds(g * 16, 16)]
                for e in range(16):
                    w16 = _bcast_lane(w_g, e)
                    r = g * 16 + e
                    for j in range(8):
                        rows[r, pl.ds(j * 16, 16)] = (
                            rows[r, pl.ds(j * 16, 16)] * w16)
                return carry2

            lax.fori_loop(0, CHUNK // 16, grp_body, 0)

        def scatter(c, rows):
            pltpu.sync_copy(rows, acc.at[dst_v.at[c]], add=True)

        start_gather(0, rows_a, sem_a, ew_a, sew_a)

        def pair_body(p, carry):
            c0 = 2 * p
            start_gather(c0 + 1, rows_b, sem_b, ew_b, sew_b)
            wait_gather(rows_a, sem_a, ew_a, sew_a)
            scale(c0, rows_a, ew_a)
            scatter(c0, rows_a)
            start_gather(c0 + 2, rows_a, sem_a, ew_a, sew_a)
            wait_gather(rows_b, sem_b, ew_b, sew_b)
            scale(c0 + 1, rows_b, ew_b)
            scatter(c0 + 1, rows_b)
            return carry

        lax.fori_loop(0, (NCH - 1) // 2, pair_body, 0)
        wait_gather(rows_a, sem_a, ew_a, sew_a)
        scale(NCH - 1, rows_a, ew_a)
        scatter(NCH - 1, rows_a)
        plsc.subcore_barrier()
        for kk in range(SPT // ZR):
            def wcopy(kk=kk):
                r0 = row0 + kk * ZR
                pltpu.sync_copy(acc.at[pl.ds(r0, ZR)],
                                out_hbm.at[cid, pl.ds(r0, ZR)])
            if kk < 5:
                wcopy()
            else:
                pl.when(not_last)(wcopy)

    return k(h, src, dst2d, ew)


def _emb_body(x_ref, w_ref, b_ref, o_ref):
    o_ref[...] = (jnp.dot(x_ref[...], w_ref[...],
                          preferred_element_type=jnp.float32) + b_ref[...])


def _tc_emb(x, W, b):
    return pl.pallas_call(
        _emb_body,
        grid=(N // BLK,),
        in_specs=[pl.BlockSpec((BLK, D), lambda i: (i, 0)),
                  pl.BlockSpec((D, H), lambda i: (0, 0)),
                  pl.BlockSpec((1, H), lambda i: (0, 0))],
        out_specs=pl.BlockSpec((BLK, H), lambda i: (i, 0)),
        out_shape=jax.ShapeDtypeStruct((N, H), jnp.float32),
    )(x, W, b.reshape(1, H))


def _mlp_body(h_ref, a0_ref, a1_ref, w1_ref, b1_ref, w2_ref, b2_ref, o_ref):
    z = h_ref[...] + a0_ref[0] + a1_ref[0]
    t = jnp.maximum(jnp.dot(z, w1_ref[...],
                            preferred_element_type=jnp.float32) + b1_ref[...], 0.0)
    o_ref[...] = jnp.maximum(
        jnp.dot(t, w2_ref[...], preferred_element_type=jnp.float32)
        + b2_ref[...], 0.0)


def _tc_mlp(h, agg, W1, b1, W2, b2):
    nb = N // BLK
    return pl.pallas_call(
        _mlp_body,
        grid=(nb,),
        in_specs=[pl.BlockSpec((BLK, H), lambda i: (i, 0)),
                  pl.BlockSpec((1, BLK, H), lambda i: (0, i, 0)),
                  pl.BlockSpec((1, BLK, H), lambda i: (1, i, 0)),
                  pl.BlockSpec((H, H), lambda i: (0, 0)),
                  pl.BlockSpec((1, H), lambda i: (0, 0)),
                  pl.BlockSpec((H, H), lambda i: (0, 0)),
                  pl.BlockSpec((1, H), lambda i: (0, 0))],
        out_specs=pl.BlockSpec((BLK, H), lambda i: (i, 0)),
        out_shape=jax.ShapeDtypeStruct((N, H), jnp.float32),
    )(h, agg, agg, W1, b1.reshape(1, H), W2, b2.reshape(1, H))


def _mlp_ro_body(h_ref, a0_ref, a1_ref, w1_ref, b1_ref, w2_ref, b2_ref,
                 wr_ref, br_ref, o_ref):
    z = h_ref[...] + a0_ref[0] + a1_ref[0]
    t = jnp.maximum(jnp.dot(z, w1_ref[...],
                            preferred_element_type=jnp.float32) + b1_ref[...], 0.0)
    u = jnp.maximum(
        jnp.dot(t, w2_ref[...], preferred_element_type=jnp.float32)
        + b2_ref[...], 0.0)
    o_ref[...] = (jnp.dot(u, wr_ref[...],
                          preferred_element_type=jnp.float32) + br_ref[...])


def _tc_mlp_ro(h, agg, W1, b1, W2, b2, Wr, br):
    nb = N // BLK
    return pl.pallas_call(
        _mlp_ro_body,
        grid=(nb,),
        in_specs=[pl.BlockSpec((BLK, H), lambda i: (i, 0)),
                  pl.BlockSpec((1, BLK, H), lambda i: (0, i, 0)),
                  pl.BlockSpec((1, BLK, H), lambda i: (1, i, 0)),
                  pl.BlockSpec((H, H), lambda i: (0, 0)),
                  pl.BlockSpec((1, H), lambda i: (0, 0)),
                  pl.BlockSpec((H, H), lambda i: (0, 0)),
                  pl.BlockSpec((1, H), lambda i: (0, 0)),
                  pl.BlockSpec((H, OUT), lambda i: (0, 0)),
                  pl.BlockSpec((1, OUT), lambda i: (0, 0))],
        out_specs=pl.BlockSpec((BLK, OUT), lambda i: (i, 0)),
        out_shape=jax.ShapeDtypeStruct((N, OUT), jnp.float32),
    )(h, agg, agg, W1, b1.reshape(1, H), W2, b2.reshape(1, H),
      Wr, br.reshape(1, OUT))


def kernel(features, edge_index, edge_weight,
           W_emb, b_emb,
           W1_0, b1_0, W2_0, b2_0,
           W1_1, b1_1, W2_1, b2_1,
           W_ro, b_ro):
    src = edge_index[0].astype(jnp.int32)
    dst2d = edge_index[1].astype(jnp.int32).reshape(NW, NCH, CHUNK)
    h0 = _tc_emb(features, W_emb, b_emb)
    agg0 = _sc_agg(h0, src, dst2d, edge_weight)
    h1 = _tc_mlp(h0, agg0, W1_0, b1_0, W2_0, b2_0)
    agg1 = _sc_agg(h1, src, dst2d, edge_weight)
    return _tc_mlp_ro(h1, agg1, W1_1, b1_1, W2_1, b2_1, W_ro, b_ro)


# speedup vs baseline: 1.3835x; 1.0153x over previous
import functools

import jax
import jax.numpy as jnp
from jax import lax
from jax.experimental import pallas as pl
from jax.experimental.pallas import tpu as pltpu
from jax.experimental.pallas import tpu_sc as plsc

N = 10000
E = 320000
D = 128
H = 128
OUT = 128

NW = 32
EPW = E // NW
CHUNK = 80
NCH = EPW // CHUNK
SPT = 640
ZR = 80
BLK = 1000

_GATHER_DNUMS = lax.GatherDimensionNumbers(
    offset_dims=(), collapsed_slice_dims=(0,), start_index_map=(0,))


def _bcast_lane(v, e):
    idx = jnp.full((16, 1), e, jnp.int32)
    return lax.gather(v, idx, _GATHER_DNUMS, slice_sizes=(1,),
                      mode=lax.GatherScatterMode.PROMISE_IN_BOUNDS)


def _sc_agg(h, src, dst2d, ew):
    mesh = plsc.VectorSubcoreMesh(core_axis_name="c", subcore_axis_name="s")

    @functools.partial(
        pl.kernel,
        out_type=jax.ShapeDtypeStruct((2, N, H), jnp.float32),
        mesh=mesh,
        scratch_types=[
            pltpu.VMEM_SHARED((N, H), jnp.float32),
            pltpu.VMEM((EPW,), jnp.int32),
            pltpu.VMEM((NCH, CHUNK), jnp.int32),
            pltpu.VMEM((CHUNK, H), jnp.float32),
            pltpu.VMEM((CHUNK, H), jnp.float32),
            pltpu.VMEM((CHUNK,), jnp.float32),
            pltpu.VMEM((CHUNK,), jnp.float32),
            pltpu.SemaphoreType.DMA,
            pltpu.SemaphoreType.DMA,
            pltpu.SemaphoreType.DMA,
            pltpu.SemaphoreType.DMA,
        ],
    )
    def k(h_hbm, src_hbm, dst_hbm, ew_hbm, out_hbm,
          acc, src_v, dst_v, rows_a, rows_b, ew_a, ew_b,
          sem_a, sem_b, sew_a, sew_b):
        cid = lax.axis_index("c")
        sid = lax.axis_index("s")
        wid = cid * 16 + sid
        ebase = wid * EPW
        pltpu.async_copy(src_hbm.at[pl.ds(ebase, EPW)], src_v, sem_a)
        pltpu.async_copy(dst_hbm.at[wid], dst_v, sem_b)

        z16 = jnp.zeros((16,), jnp.float32)

        def zero_body(r, carry):
            for j in range(8):
                rows_a[r, pl.ds(j * 16, 16)] = z16
            return carry

        lax.fori_loop(0, ZR, zero_body, 0)
        row0 = sid * SPT
        not_last = sid != 15
        for kk in range(SPT // ZR):
            def zfire(kk=kk):
                pltpu.async_copy(rows_a, acc.at[pl.ds(row0 + kk * ZR, ZR)],
                                 sew_a)
            if kk < 5:
                zfire()
            else:
                pl.when(not_last)(zfire)
        for kk in range(SPT // ZR):
            def zdrain(kk=kk):
                pltpu.make_async_copy(
                    rows_a, acc.at[pl.ds(row0, ZR)], sew_a).wait()
            if kk < 5:
                zdrain()
            else:
                pl.when(not_last)(zdrain)
        pltpu.make_async_copy(
            src_hbm.at[pl.ds(ebase, EPW)], src_v, sem_a).wait()
        pltpu.make_async_copy(dst_hbm.at[wid], dst_v, sem_b).wait()
        plsc.subcore_barrier()

        def start_gather(c, rows, sem, ew, sew):
            pltpu.async_copy(ew_hbm.at[pl.ds(ebase + c * CHUNK, CHUNK)],
                             ew, sew)
            pltpu.async_copy(
                h_hbm.at[src_v.at[pl.ds(c * CHUNK, CHUNK)]], rows, sem)

        def wait_gather(rows, sem, ew, sew):
            pltpu.make_async_copy(ew_hbm.at[pl.ds(0, CHUNK)], ew, sew).wait()
            pltpu.make_async_copy(
                h_hbm.at[src_v.at[pl.ds(0, CHUNK)]], rows, sem).wait()

        def scale(c, rows, ew):
            def grp_body(g, carry2):
                w_g = ew[pl.ds(g * 16, 16)]
                for e in range(16):
                    w16 = _bcast_lane(w_g, e)
                    r = g * 16 + e
                    for j in range(8):
                        rows[r, pl.ds(j * 16, 16)] = (
                            rows[r, pl.ds(j * 16, 16)] * w16)
                return carry2

            lax.fori_loop(0, CHUNK // 16, grp_body, 0)

        def scatter(c, rows):
            pltpu.sync_copy(rows, acc.at[dst_v.at[c]], add=True)

        start_gather(0, rows_a, sem_a, ew_a, sew_a)

        def pair_body(p, carry):
            c0 = 2 * p
            start_gather(c0 + 1, rows_b, sem_b, ew_b, sew_b)
            wait_gather(rows_a, sem_a, ew_a, sew_a)
            scale(c0, rows_a, ew_a)
            scatter(c0, rows_a)
            start_gather(c0 + 2, rows_a, sem_a, ew_a, sew_a)
            wait_gather(rows_b, sem_b, ew_b, sew_b)
            scale(c0 + 1, rows_b, ew_b)
            scatter(c0 + 1, rows_b)
            return carry

        lax.fori_loop(0, (NCH - 1) // 2, pair_body, 0)
        wait_gather(rows_a, sem_a, ew_a, sew_a)
        scale(NCH - 1, rows_a, ew_a)
        scatter(NCH - 1, rows_a)
        plsc.subcore_barrier()

        def wfull():
            pltpu.sync_copy(acc.at[pl.ds(row0, SPT)],
                            out_hbm.at[cid, pl.ds(row0, SPT)])

        def wlast():
            pltpu.sync_copy(acc.at[pl.ds(row0, 400)],
                            out_hbm.at[cid, pl.ds(row0, 400)])

        pl.when(not_last)(wfull)
        pl.when(jnp.logical_not(not_last))(wlast)

    return k(h, src, dst2d, ew)


def _emb_body(x_ref, w_ref, b_ref, o_ref):
    o_ref[...] = (jnp.dot(x_ref[...], w_ref[...],
                          preferred_element_type=jnp.float32) + b_ref[...])


def _tc_emb(x, W, b):
    return pl.pallas_call(
        _emb_body,
        grid=(N // BLK,),
        in_specs=[pl.BlockSpec((BLK, D), lambda i: (i, 0)),
                  pl.BlockSpec((D, H), lambda i: (0, 0)),
                  pl.BlockSpec((1, H), lambda i: (0, 0))],
        out_specs=pl.BlockSpec((BLK, H), lambda i: (i, 0)),
        out_shape=jax.ShapeDtypeStruct((N, H), jnp.float32),
    )(x, W, b.reshape(1, H))


def _mlp_body(h_ref, a0_ref, a1_ref, w1_ref, b1_ref, w2_ref, b2_ref, o_ref):
    z = h_ref[...] + a0_ref[0] + a1_ref[0]
    t = jnp.maximum(jnp.dot(z, w1_ref[...],
                            preferred_element_type=jnp.float32) + b1_ref[...], 0.0)
    o_ref[...] = jnp.maximum(
        jnp.dot(t, w2_ref[...], preferred_element_type=jnp.float32)
        + b2_ref[...], 0.0)


def _tc_mlp(h, agg, W1, b1, W2, b2):
    nb = N // BLK
    return pl.pallas_call(
        _mlp_body,
        grid=(nb,),
        in_specs=[pl.BlockSpec((BLK, H), lambda i: (i, 0)),
                  pl.BlockSpec((1, BLK, H), lambda i: (0, i, 0)),
                  pl.BlockSpec((1, BLK, H), lambda i: (1, i, 0)),
                  pl.BlockSpec((H, H), lambda i: (0, 0)),
                  pl.BlockSpec((1, H), lambda i: (0, 0)),
                  pl.BlockSpec((H, H), lambda i: (0, 0)),
                  pl.BlockSpec((1, H), lambda i: (0, 0))],
        out_specs=pl.BlockSpec((BLK, H), lambda i: (i, 0)),
        out_shape=jax.ShapeDtypeStruct((N, H), jnp.float32),
    )(h, agg, agg, W1, b1.reshape(1, H), W2, b2.reshape(1, H))


def _mlp_ro_body(h_ref, a0_ref, a1_ref, w1_ref, b1_ref, w2_ref, b2_ref,
                 wr_ref, br_ref, o_ref):
    z = h_ref[...] + a0_ref[0] + a1_ref[0]
    t = jnp.maximum(jnp.dot(z, w1_ref[...],
                            preferred_element_type=jnp.float32) + b1_ref[...], 0.0)
    u = jnp.maximum(
        jnp.dot(t, w2_ref[...], preferred_element_type=jnp.float32)
        + b2_ref[...], 0.0)
    o_ref[...] = (jnp.dot(u, wr_ref[...],
                          preferred_element_type=jnp.float32) + br_ref[...])


def _tc_mlp_ro(h, agg, W1, b1, W2, b2, Wr, br):
    nb = N // BLK
    return pl.pallas_call(
        _mlp_ro_body,
        grid=(nb,),
        in_specs=[pl.BlockSpec((BLK, H), lambda i: (i, 0)),
                  pl.BlockSpec((1, BLK, H), lambda i: (0, i, 0)),
                  pl.BlockSpec((1, BLK, H), lambda i: (1, i, 0)),
                  pl.BlockSpec((H, H), lambda i: (0, 0)),
                  pl.BlockSpec((1, H), lambda i: (0, 0)),
                  pl.BlockSpec((H, H), lambda i: (0, 0)),
                  pl.BlockSpec((1, H), lambda i: (0, 0)),
                  pl.BlockSpec((H, OUT), lambda i: (0, 0)),
                  pl.BlockSpec((1, OUT), lambda i: (0, 0))],
        out_specs=pl.BlockSpec((BLK, OUT), lambda i: (i, 0)),
        out_shape=jax.ShapeDtypeStruct((N, OUT), jnp.float32),
    )(h, agg, agg, W1, b1.reshape(1, H), W2, b2.reshape(1, H),
      Wr, br.reshape(1, OUT))


def kernel(features, edge_index, edge_weight,
           W_emb, b_emb,
           W1_0, b1_0, W2_0, b2_0,
           W1_1, b1_1, W2_1, b2_1,
           W_ro, b_ro):
    src = edge_index[0].astype(jnp.int32)
    dst2d = edge_index[1].astype(jnp.int32).reshape(NW, NCH, CHUNK)
    h0 = _tc_emb(features, W_emb, b_emb)
    agg0 = _sc_agg(h0, src, dst2d, edge_weight)
    h1 = _tc_mlp(h0, agg0, W1_0, b1_0, W2_0, b2_0)
    agg1 = _sc_agg(h1, src, dst2d, edge_weight)
    return _tc_mlp_ro(h1, agg1, W1_1, b1_1, W2_1, b2_1, W_ro, b_ro)


# speedup vs baseline: 1.4177x; 1.0247x over previous
import functools

import jax
import jax.numpy as jnp
from jax import lax
from jax.experimental import pallas as pl
from jax.experimental.pallas import tpu as pltpu
from jax.experimental.pallas import tpu_sc as plsc

N = 10000
E = 320000
D = 128
H = 128
OUT = 128

NW = 32
EPW = E // NW
CHUNK = 80
NCH = EPW // CHUNK
SPT = 640
ZR = 80
BLK = 2000

_GATHER_DNUMS = lax.GatherDimensionNumbers(
    offset_dims=(), collapsed_slice_dims=(0,), start_index_map=(0,))


def _bcast_lane(v, e):
    idx = jnp.full((16, 1), e, jnp.int32)
    return lax.gather(v, idx, _GATHER_DNUMS, slice_sizes=(1,),
                      mode=lax.GatherScatterMode.PROMISE_IN_BOUNDS)


def _sc_agg(h, src, dst2d, ew):
    mesh = plsc.VectorSubcoreMesh(core_axis_name="c", subcore_axis_name="s")

    @functools.partial(
        pl.kernel,
        out_type=jax.ShapeDtypeStruct((2, N, H), jnp.float32),
        mesh=mesh,
        scratch_types=[
            pltpu.VMEM_SHARED((N, H), jnp.float32),
            pltpu.VMEM((EPW,), jnp.int32),
            pltpu.VMEM((NCH, CHUNK), jnp.int32),
            pltpu.VMEM((CHUNK, H), jnp.float32),
            pltpu.VMEM((CHUNK, H), jnp.float32),
            pltpu.VMEM((CHUNK,), jnp.float32),
            pltpu.VMEM((CHUNK,), jnp.float32),
            pltpu.SemaphoreType.DMA,
            pltpu.SemaphoreType.DMA,
            pltpu.SemaphoreType.DMA,
            pltpu.SemaphoreType.DMA,
        ],
    )
    def k(h_hbm, src_hbm, dst_hbm, ew_hbm, out_hbm,
          acc, src_v, dst_v, rows_a, rows_b, ew_a, ew_b,
          sem_a, sem_b, sew_a, sew_b):
        cid = lax.axis_index("c")
        sid = lax.axis_index("s")
        wid = cid * 16 + sid
        ebase = wid * EPW
        pltpu.async_copy(src_hbm.at[pl.ds(ebase, EPW)], src_v, sem_a)
        pltpu.async_copy(dst_hbm.at[wid], dst_v, sem_b)

        z16 = jnp.zeros((16,), jnp.float32)

        def zero_body(r, carry):
            for j in range(8):
                rows_a[r, pl.ds(j * 16, 16)] = z16
            return carry

        lax.fori_loop(0, ZR, zero_body, 0)
        row0 = sid * SPT
        not_last = sid != 15
        for kk in range(SPT // ZR):
            def zfire(kk=kk):
                pltpu.async_copy(rows_a, acc.at[pl.ds(row0 + kk * ZR, ZR)],
                                 sew_a)
            if kk < 5:
                zfire()
            else:
                pl.when(not_last)(zfire)
        for kk in range(SPT // ZR):
            def zdrain(kk=kk):
                pltpu.make_async_copy(
                    rows_a, acc.at[pl.ds(row0, ZR)], sew_a).wait()
            if kk < 5:
                zdrain()
            else:
                pl.when(not_last)(zdrain)
        pltpu.make_async_copy(
            src_hbm.at[pl.ds(ebase, EPW)], src_v, sem_a).wait()
        pltpu.make_async_copy(dst_hbm.at[wid], dst_v, sem_b).wait()
        plsc.subcore_barrier()

        def start_gather(c, rows, sem, ew, sew):
            pltpu.async_copy(ew_hbm.at[pl.ds(ebase + c * CHUNK, CHUNK)],
                             ew, sew)
            pltpu.async_copy(
                h_hbm.at[src_v.at[pl.ds(c * CHUNK, CHUNK)]], rows, sem)

        def wait_gather(rows, sem, ew, sew):
            pltpu.make_async_copy(ew_hbm.at[pl.ds(0, CHUNK)], ew, sew).wait()
            pltpu.make_async_copy(
                h_hbm.at[src_v.at[pl.ds(0, CHUNK)]], rows, sem).wait()

        def scale(c, rows, ew):
            def grp_body(g, carry2):
                w_g = ew[pl.ds(g * 16, 16)]
                for e in range(16):
                    w16 = _bcast_lane(w_g, e)
                    r = g * 16 + e
                    for j in range(8):
                        rows[r, pl.ds(j * 16, 16)] = (
                            rows[r, pl.ds(j * 16, 16)] * w16)
                return carry2

            lax.fori_loop(0, CHUNK // 16, grp_body, 0)

        def scatter(c, rows):
            pltpu.sync_copy(rows, acc.at[dst_v.at[c]], add=True)

        start_gather(0, rows_a, sem_a, ew_a, sew_a)

        def pair_body(p, carry):
            c0 = 2 * p
            start_gather(c0 + 1, rows_b, sem_b, ew_b, sew_b)
            wait_gather(rows_a, sem_a, ew_a, sew_a)
            scale(c0, rows_a, ew_a)
            scatter(c0, rows_a)
            start_gather(c0 + 2, rows_a, sem_a, ew_a, sew_a)
            wait_gather(rows_b, sem_b, ew_b, sew_b)
            scale(c0 + 1, rows_b, ew_b)
            scatter(c0 + 1, rows_b)
            return carry

        lax.fori_loop(0, (NCH - 1) // 2, pair_body, 0)
        wait_gather(rows_a, sem_a, ew_a, sew_a)
        scale(NCH - 1, rows_a, ew_a)
        scatter(NCH - 1, rows_a)
        plsc.subcore_barrier()

        def wfull():
            pltpu.sync_copy(acc.at[pl.ds(row0, SPT)],
                            out_hbm.at[cid, pl.ds(row0, SPT)])

        def wlast():
            pltpu.sync_copy(acc.at[pl.ds(row0, 400)],
                            out_hbm.at[cid, pl.ds(row0, 400)])

        pl.when(not_last)(wfull)
        pl.when(jnp.logical_not(not_last))(wlast)

    return k(h, src, dst2d, ew)


def _emb_body(x_ref, w_ref, b_ref, o_ref):
    o_ref[...] = (jnp.dot(x_ref[...], w_ref[...],
                          preferred_element_type=jnp.float32) + b_ref[...])


def _tc_emb(x, W, b):
    return pl.pallas_call(
        _emb_body,
        grid=(N // BLK,),
        in_specs=[pl.BlockSpec((BLK, D), lambda i: (i, 0)),
                  pl.BlockSpec((D, H), lambda i: (0, 0)),
                  pl.BlockSpec((1, H), lambda i: (0, 0))],
        out_specs=pl.BlockSpec((BLK, H), lambda i: (i, 0)),
        out_shape=jax.ShapeDtypeStruct((N, H), jnp.float32),
    )(x, W, b.reshape(1, H))


def _mlp_body(h_ref, a0_ref, a1_ref, w1_ref, b1_ref, w2_ref, b2_ref, o_ref):
    z = h_ref[...] + a0_ref[0] + a1_ref[0]
    t = jnp.maximum(jnp.dot(z, w1_ref[...],
                            preferred_element_type=jnp.float32) + b1_ref[...], 0.0)
    o_ref[...] = jnp.maximum(
        jnp.dot(t, w2_ref[...], preferred_element_type=jnp.float32)
        + b2_ref[...], 0.0)


def _tc_mlp(h, agg, W1, b1, W2, b2):
    nb = N // BLK
    return pl.pallas_call(
        _mlp_body,
        grid=(nb,),
        in_specs=[pl.BlockSpec((BLK, H), lambda i: (i, 0)),
                  pl.BlockSpec((1, BLK, H), lambda i: (0, i, 0)),
                  pl.BlockSpec((1, BLK, H), lambda i: (1, i, 0)),
                  pl.BlockSpec((H, H), lambda i: (0, 0)),
                  pl.BlockSpec((1, H), lambda i: (0, 0)),
                  pl.BlockSpec((H, H), lambda i: (0, 0)),
                  pl.BlockSpec((1, H), lambda i: (0, 0))],
        out_specs=pl.BlockSpec((BLK, H), lambda i: (i, 0)),
        out_shape=jax.ShapeDtypeStruct((N, H), jnp.float32),
    )(h, agg, agg, W1, b1.reshape(1, H), W2, b2.reshape(1, H))


def _mlp_ro_body(h_ref, a0_ref, a1_ref, w1_ref, b1_ref, w2_ref, b2_ref,
                 wr_ref, br_ref, o_ref):
    z = h_ref[...] + a0_ref[0] + a1_ref[0]
    t = jnp.maximum(jnp.dot(z, w1_ref[...],
                            preferred_element_type=jnp.float32) + b1_ref[...], 0.0)
    u = jnp.maximum(
        jnp.dot(t, w2_ref[...], preferred_element_type=jnp.float32)
        + b2_ref[...], 0.0)
    o_ref[...] = (jnp.dot(u, wr_ref[...],
                          preferred_element_type=jnp.float32) + br_ref[...])


def _tc_mlp_ro(h, agg, W1, b1, W2, b2, Wr, br):
    nb = N // BLK
    return pl.pallas_call(
        _mlp_ro_body,
        grid=(nb,),
        in_specs=[pl.BlockSpec((BLK, H), lambda i: (i, 0)),
                  pl.BlockSpec((1, BLK, H), lambda i: (0, i, 0)),
                  pl.BlockSpec((1, BLK, H), lambda i: (1, i, 0)),
                  pl.BlockSpec((H, H), lambda i: (0, 0)),
                  pl.BlockSpec((1, H), lambda i: (0, 0)),
                  pl.BlockSpec((H, H), lambda i: (0, 0)),
                  pl.BlockSpec((1, H), lambda i: (0, 0)),
                  pl.BlockSpec((H, OUT), lambda i: (0, 0)),
                  pl.BlockSpec((1, OUT), lambda i: (0, 0))],
        out_specs=pl.BlockSpec((BLK, OUT), lambda i: (i, 0)),
        out_shape=jax.ShapeDtypeStruct((N, OUT), jnp.float32),
    )(h, agg, agg, W1, b1.reshape(1, H), W2, b2.reshape(1, H),
      Wr, br.reshape(1, OUT))


def kernel(features, edge_index, edge_weight,
           W_emb, b_emb,
           W1_0, b1_0, W2_0, b2_0,
           W1_1, b1_1, W2_1, b2_1,
           W_ro, b_ro):
    src = edge_index[0].astype(jnp.int32)
    dst2d = edge_index[1].astype(jnp.int32).reshape(NW, NCH, CHUNK)
    h0 = _tc_emb(features, W_emb, b_emb)
    agg0 = _sc_agg(h0, src, dst2d, edge_weight)
    h1 = _tc_mlp(h0, agg0, W1_0, b1_0, W2_0, b2_0)
    agg1 = _sc_agg(h1, src, dst2d, edge_weight)
    return _tc_mlp_ro(h1, agg1, W1_1, b1_1, W2_1, b2_1, W_ro, b_ro)


# speedup vs baseline: 1.4336x; 1.0112x over previous
import functools

import jax
import jax.numpy as jnp
from jax import lax
from jax.experimental import pallas as pl
from jax.experimental.pallas import tpu as pltpu
from jax.experimental.pallas import tpu_sc as plsc

N = 10000
E = 320000
D = 128
H = 128
OUT = 128

NW = 32
EPW = E // NW
CHUNK = 80
NCH = EPW // CHUNK
SPT = 640
ZR = 80
BLK = 5000

_GATHER_DNUMS = lax.GatherDimensionNumbers(
    offset_dims=(), collapsed_slice_dims=(0,), start_index_map=(0,))


def _bcast_lane(v, e):
    idx = jnp.full((16, 1), e, jnp.int32)
    return lax.gather(v, idx, _GATHER_DNUMS, slice_sizes=(1,),
                      mode=lax.GatherScatterMode.PROMISE_IN_BOUNDS)


def _sc_agg(h, src, dst2d, ew):
    mesh = plsc.VectorSubcoreMesh(core_axis_name="c", subcore_axis_name="s")

    @functools.partial(
        pl.kernel,
        out_type=jax.ShapeDtypeStruct((2, N, H), jnp.float32),
        mesh=mesh,
        scratch_types=[
            pltpu.VMEM_SHARED((N, H), jnp.float32),
            pltpu.VMEM((EPW,), jnp.int32),
            pltpu.VMEM((NCH, CHUNK), jnp.int32),
            pltpu.VMEM((CHUNK, H), jnp.float32),
            pltpu.VMEM((CHUNK, H), jnp.float32),
            pltpu.VMEM((CHUNK,), jnp.float32),
            pltpu.VMEM((CHUNK,), jnp.float32),
            pltpu.SemaphoreType.DMA,
            pltpu.SemaphoreType.DMA,
            pltpu.SemaphoreType.DMA,
            pltpu.SemaphoreType.DMA,
        ],
    )
    def k(h_hbm, src_hbm, dst_hbm, ew_hbm, out_hbm,
          acc, src_v, dst_v, rows_a, rows_b, ew_a, ew_b,
          sem_a, sem_b, sew_a, sew_b):
        cid = lax.axis_index("c")
        sid = lax.axis_index("s")
        wid = cid * 16 + sid
        ebase = wid * EPW
        pltpu.async_copy(src_hbm.at[pl.ds(ebase, EPW)], src_v, sem_a)
        pltpu.async_copy(dst_hbm.at[wid], dst_v, sem_b)

        z16 = jnp.zeros((16,), jnp.float32)

        def zero_body(r, carry):
            for j in range(8):
                rows_a[r, pl.ds(j * 16, 16)] = z16
            return carry

        lax.fori_loop(0, ZR, zero_body, 0)
        row0 = sid * SPT
        not_last = sid != 15
        for kk in range(SPT // ZR):
            def zfire(kk=kk):
                pltpu.async_copy(rows_a, acc.at[pl.ds(row0 + kk * ZR, ZR)],
                                 sew_a)
            if kk < 5:
                zfire()
            else:
                pl.when(not_last)(zfire)
        for kk in range(SPT // ZR):
            def zdrain(kk=kk):
                pltpu.make_async_copy(
                    rows_a, acc.at[pl.ds(row0, ZR)], sew_a).wait()
            if kk < 5:
                zdrain()
            else:
                pl.when(not_last)(zdrain)
        pltpu.make_async_copy(
            src_hbm.at[pl.ds(ebase, EPW)], src_v, sem_a).wait()
        pltpu.make_async_copy(dst_hbm.at[wid], dst_v, sem_b).wait()
        plsc.subcore_barrier()

        def start_gather(c, rows, sem, ew, sew):
            pltpu.async_copy(ew_hbm.at[pl.ds(ebase + c * CHUNK, CHUNK)],
                             ew, sew)
            pltpu.async_copy(
                h_hbm.at[src_v.at[pl.ds(c * CHUNK, CHUNK)]], rows, sem)

        def wait_gather(rows, sem, ew, sew):
            pltpu.make_async_copy(ew_hbm.at[pl.ds(0, CHUNK)], ew, sew).wait()
            pltpu.make_async_copy(
                h_hbm.at[src_v.at[pl.ds(0, CHUNK)]], rows, sem).wait()

        def scale(c, rows, ew):
            def grp_body(g, carry2):
                w_g = ew[pl.ds(g * 16, 16)]
                for e in range(16):
                    w16 = _bcast_lane(w_g, e)
                    r = g * 16 + e
                    for j in range(8):
                        rows[r, pl.ds(j * 16, 16)] = (
                            rows[r, pl.ds(j * 16, 16)] * w16)
                return carry2

            lax.fori_loop(0, CHUNK // 16, grp_body, 0)

        def scatter(c, rows):
            pltpu.sync_copy(rows, acc.at[dst_v.at[c]], add=True)

        start_gather(0, rows_a, sem_a, ew_a, sew_a)

        def pair_body(p, carry):
            c0 = 2 * p
            start_gather(c0 + 1, rows_b, sem_b, ew_b, sew_b)
            wait_gather(rows_a, sem_a, ew_a, sew_a)
            scale(c0, rows_a, ew_a)
            scatter(c0, rows_a)
            start_gather(c0 + 2, rows_a, sem_a, ew_a, sew_a)
            wait_gather(rows_b, sem_b, ew_b, sew_b)
            scale(c0 + 1, rows_b, ew_b)
            scatter(c0 + 1, rows_b)
            return carry

        lax.fori_loop(0, (NCH - 1) // 2, pair_body, 0)
        wait_gather(rows_a, sem_a, ew_a, sew_a)
        scale(NCH - 1, rows_a, ew_a)
        scatter(NCH - 1, rows_a)
        plsc.subcore_barrier()

        def wfull():
            pltpu.sync_copy(acc.at[pl.ds(row0, SPT)],
                            out_hbm.at[cid, pl.ds(row0, SPT)])

        def wlast():
            pltpu.sync_copy(acc.at[pl.ds(row0, 400)],
                            out_hbm.at[cid, pl.ds(row0, 400)])

        pl.when(not_last)(wfull)
        pl.when(jnp.logical_not(not_last))(wlast)

    return k(h, src, dst2d, ew)


def _emb_body(x_ref, w_ref, b_ref, o_ref):
    o_ref[...] = (jnp.dot(x_ref[...], w_ref[...],
                          preferred_element_type=jnp.float32) + b_ref[...])


def _tc_emb(x, W, b):
    return pl.pallas_call(
        _emb_body,
        grid=(N // BLK,),
        in_specs=[pl.BlockSpec((BLK, D), lambda i: (i, 0)),
                  pl.BlockSpec((D, H), lambda i: (0, 0)),
                  pl.BlockSpec((1, H), lambda i: (0, 0))],
        out_specs=pl.BlockSpec((BLK, H), lambda i: (i, 0)),
        out_shape=jax.ShapeDtypeStruct((N, H), jnp.float32),
    )(x, W, b.reshape(1, H))


def _mlp_body(h_ref, a0_ref, a1_ref, w1_ref, b1_ref, w2_ref, b2_ref, o_ref):
    z = h_ref[...] + a0_ref[0] + a1_ref[0]
    t = jnp.maximum(jnp.dot(z, w1_ref[...],
                            preferred_element_type=jnp.float32) + b1_ref[...], 0.0)
    o_ref[...] = jnp.maximum(
        jnp.dot(t, w2_ref[...], preferred_element_type=jnp.float32)
        + b2_ref[...], 0.0)


def _tc_mlp(h, agg, W1, b1, W2, b2):
    nb = N // BLK
    return pl.pallas_call(
        _mlp_body,
        grid=(nb,),
        in_specs=[pl.BlockSpec((BLK, H), lambda i: (i, 0)),
                  pl.BlockSpec((1, BLK, H), lambda i: (0, i, 0)),
                  pl.BlockSpec((1, BLK, H), lambda i: (1, i, 0)),
                  pl.BlockSpec((H, H), lambda i: (0, 0)),
                  pl.BlockSpec((1, H), lambda i: (0, 0)),
                  pl.BlockSpec((H, H), lambda i: (0, 0)),
                  pl.BlockSpec((1, H), lambda i: (0, 0))],
        out_specs=pl.BlockSpec((BLK, H), lambda i: (i, 0)),
        out_shape=jax.ShapeDtypeStruct((N, H), jnp.float32),
    )(h, agg, agg, W1, b1.reshape(1, H), W2, b2.reshape(1, H))


def _mlp_ro_body(h_ref, a0_ref, a1_ref, w1_ref, b1_ref, w2_ref, b2_ref,
                 wr_ref, br_ref, o_ref):
    z = h_ref[...] + a0_ref[0] + a1_ref[0]
    t = jnp.maximum(jnp.dot(z, w1_ref[...],
                            preferred_element_type=jnp.float32) + b1_ref[...], 0.0)
    u = jnp.maximum(
        jnp.dot(t, w2_ref[...], preferred_element_type=jnp.float32)
        + b2_ref[...], 0.0)
    o_ref[...] = (jnp.dot(u, wr_ref[...],
                          preferred_element_type=jnp.float32) + br_ref[...])


def _tc_mlp_ro(h, agg, W1, b1, W2, b2, Wr, br):
    nb = N // BLK
    return pl.pallas_call(
        _mlp_ro_body,
        grid=(nb,),
        in_specs=[pl.BlockSpec((BLK, H), lambda i: (i, 0)),
                  pl.BlockSpec((1, BLK, H), lambda i: (0, i, 0)),
                  pl.BlockSpec((1, BLK, H), lambda i: (1, i, 0)),
                  pl.BlockSpec((H, H), lambda i: (0, 0)),
                  pl.BlockSpec((1, H), lambda i: (0, 0)),
                  pl.BlockSpec((H, H), lambda i: (0, 0)),
                  pl.BlockSpec((1, H), lambda i: (0, 0)),
                  pl.BlockSpec((H, OUT), lambda i: (0, 0)),
                  pl.BlockSpec((1, OUT), lambda i: (0, 0))],
        out_specs=pl.BlockSpec((BLK, OUT), lambda i: (i, 0)),
        out_shape=jax.ShapeDtypeStruct((N, OUT), jnp.float32),
    )(h, agg, agg, W1, b1.reshape(1, H), W2, b2.reshape(1, H),
      Wr, br.reshape(1, OUT))


def kernel(features, edge_index, edge_weight,
           W_emb, b_emb,
           W1_0, b1_0, W2_0, b2_0,
           W1_1, b1_1, W2_1, b2_1,
           W_ro, b_ro):
    src = edge_index[0].astype(jnp.int32)
    dst2d = edge_index[1].astype(jnp.int32).reshape(NW, NCH, CHUNK)
    h0 = _tc_emb(features, W_emb, b_emb)
    agg0 = _sc_agg(h0, src, dst2d, edge_weight)
    h1 = _tc_mlp(h0, agg0, W1_0, b1_0, W2_0, b2_0)
    agg1 = _sc_agg(h1, src, dst2d, edge_weight)
    return _tc_mlp_ro(h1, agg1, W1_1, b1_1, W2_1, b2_1, W_ro, b_ro)
